# baseline (speedup 1.0000x reference)
import jax
import jax.numpy as jnp
from jax import lax
from jax.experimental import pallas as pl

VOCAB = 1000
VOCAB_PAD = 1024
BLOCK_R = 1600


def _onehot_block(x_ref, o_ref):
    xi = x_ref[...]
    iota = lax.broadcasted_iota(jnp.int32, (BLOCK_R, VOCAB_PAD), 1)
    o_ref[...] = (xi == iota).astype(jnp.float32)


def kernel(x):
    B, S = x.shape
    R = B * S
    xp = x.astype(jnp.int32).reshape(R, 1)
    out = pl.pallas_call(
        _onehot_block,
        grid=(R // BLOCK_R,),
        in_specs=[pl.BlockSpec((BLOCK_R, 1), lambda i: (i, 0))],
        out_specs=pl.BlockSpec((BLOCK_R, VOCAB_PAD), lambda i: (i, 0)),
        out_shape=jax.ShapeDtypeStruct((R, VOCAB_PAD), jnp.float32),
    )(xp)
    return out[:, :VOCAB].reshape(B, S, VOCAB)


# speedup vs baseline: 1.5989x; 1.5989x over previous
import jax
import jax.numpy as jnp
from jax import lax
from jax.experimental import pallas as pl

VOCAB = 1000
VOCAB_PAD = 1024
BLOCK_B = 32


def _onehot_block(x_ref, o_ref):
    xi = x_ref[...]
    iota = lax.broadcasted_iota(
        jnp.int32, (xi.shape[0], xi.shape[1], VOCAB_PAD), 2
    )
    o_ref[...] = (xi == iota).astype(jnp.float32)


def kernel(x):
    B, S = x.shape
    xp = x.astype(jnp.int32).reshape(B, S, 1)
    out = pl.pallas_call(
        _onehot_block,
        grid=(B // BLOCK_B,),
        in_specs=[pl.BlockSpec((BLOCK_B, S, 1), lambda i: (i, 0, 0))],
        out_specs=pl.BlockSpec((BLOCK_B, S, VOCAB_PAD), lambda i: (i, 0, 0)),
        out_shape=jax.ShapeDtypeStruct((B, S, VOCAB_PAD), jnp.float32),
    )(xp)
    return out[:, :, :VOCAB]
